# baseline (speedup 1.0000x reference)
import functools

import jax
import jax.numpy as jnp
from jax import lax
from jax.experimental import pallas as pl
from jax.experimental.pallas import tpu as pltpu
from jax.experimental.pallas import tpu_sc as plsc

N = 100000
E = 3200000
EMB = 16
MULT = 4
DEPTH = 2
NUMCLS = 40

NC = 2
NS = 16
NW = NC * NS
CH = 128
G = 16
NG = 49
C = G * NG
EPW = C * CH
EPAD = NW * EPW - E

H = N // 2
AH = 50176
NDUM = AH - H
TRH = AH // NS
LTR = H - (NS - 1) * TRH
ZB = 392

RB = 5000
NB = N // RB
HB = NB // 2

_mesh = plsc.VectorSubcoreMesh(core_axis_name="c", subcore_axis_name="s")
_sc_params = pltpu.CompilerParams(use_tc_tiling_on_sc=False)


def _zero_acc(zbuf, acc, s):
    def zb(i, carry):
        zbuf[i] = jnp.zeros((EMB,), jnp.float32)
        return carry
    lax.fori_loop(0, ZB, zb, None)
    base = s * TRH
    for k in range(TRH // ZB):
        pltpu.sync_copy(zbuf, acc.at[pl.ds(base + k * ZB, ZB)])


def _remap_dst(dsti_b, hoff):
    for j in range(G):
        for k in range(CH // 16):
            sl = pl.ds(k * 16, 16)
            v = dsti_b[j, sl]
            off = v - hoff
            ok = (off >= 0) & (off < H)
            dum = (H + ((j * (CH // 16) + k) % (NDUM // 16)) * 16
                   + lax.iota(jnp.int32, 16))
            dsti_b[j, sl] = jnp.where(ok, off, dum)


def _dump_acc(acc, out_hbm, c, s):
    base = s * TRH

    @pl.when(s == NS - 1)
    def _():
        pltpu.sync_copy(acc.at[pl.ds((NS - 1) * TRH, LTR)],
                        out_hbm.at[c, pl.ds((NS - 1) * TRH, LTR)])

    @pl.when(s != NS - 1)
    def _():
        pltpu.sync_copy(acc.at[pl.ds(base, TRH)],
                        out_hbm.at[c, pl.ds(base, TRH)])


def _make_degree(hoff):
    @functools.partial(
        pl.kernel,
        out_type=jax.ShapeDtypeStruct((NC, H, EMB), jnp.float32),
        mesh=_mesh,
        scratch_types=[
            pltpu.VMEM((G, CH), jnp.int32),
            pltpu.VMEM((G, CH), jnp.int32),
            pltpu.VMEM((CH, EMB), jnp.float32),
            pltpu.VMEM((ZB, EMB), jnp.float32),
            pltpu.VMEM_SHARED((AH, EMB), jnp.float32),
            pltpu.SemaphoreType.DMA,
        ],
        compiler_params=_sc_params,
    )
    def deg_kernel(dst_hbm, out_hbm, dsti0, dsti1, ones, zbuf, acc, sems):
        c = lax.axis_index("c")
        s = lax.axis_index("s")
        wid = c * NS + s
        dsti = [dsti0, dsti1]

        def ob(i, carry):
            ones[i] = jnp.ones((EMB,), jnp.float32)
            return carry
        lax.fori_loop(0, CH, ob, None)
        _zero_acc(zbuf, acc, s)
        plsc.subcore_barrier()

        def load(g, b):
            pltpu.sync_copy(dst_hbm.at[wid, pl.ds(g * G, G)], dsti[b])
            _remap_dst(dsti[b], hoff)

        def fire(b):
            for j in range(G):
                pltpu.async_copy(ones, acc.at[dsti[b].at[j]], sems, add=True)

        def drain(b):
            for j in range(G):
                pltpu.make_async_copy(ones, acc.at[dsti[b].at[j]],
                                      sems).wait()

        load(0, 0)

        def grp(k, carry):
            g = 2 * k
            fire(0)
            load(g + 1, 1)
            drain(0)
            fire(1)
            load(g + 2, 0)
            drain(1)
            return carry
        lax.fori_loop(0, (NG - 1) // 2, grp, None)
        fire(0)
        drain(0)
        plsc.subcore_barrier()
        _dump_acc(acc, out_hbm, c, s)
    return deg_kernel


_sc_degree_lo = _make_degree(0)
_sc_degree_hi = _make_degree(H)


def _make_aggregate(hoff):
    @functools.partial(
        pl.kernel,
        out_type=jax.ShapeDtypeStruct((NC, H, EMB), jnp.float32),
        mesh=_mesh,
        scratch_types=[
            pltpu.VMEM((G, CH), jnp.int32),
            pltpu.VMEM((G, CH), jnp.int32),
            pltpu.VMEM((G, CH), jnp.int32),
            pltpu.VMEM((G, CH), jnp.int32),
            pltpu.VMEM((G, CH, EMB), jnp.float32),
            pltpu.VMEM((G, CH, EMB), jnp.float32),
            pltpu.VMEM((ZB, EMB), jnp.float32),
            pltpu.VMEM_SHARED((AH, EMB), jnp.float32),
            pltpu.SemaphoreType.DMA,
            pltpu.SemaphoreType.DMA,
        ],
        compiler_params=_sc_params,
    )
    def agg_kernel(src_hbm, dst_hbm, y_hbm, out_hbm, srci0, srci1, dsti0,
                   dsti1, rows0, rows1, zbuf, acc, semg, sems):
        c = lax.axis_index("c")
        s = lax.axis_index("s")
        wid = c * NS + s
        srci = [srci0, srci1]
        dsti = [dsti0, dsti1]
        rows = [rows0, rows1]
        _zero_acc(zbuf, acc, s)
        plsc.subcore_barrier()

        def fire_gather(g, b):
            pltpu.sync_copy(src_hbm.at[wid, pl.ds(g * G, G)], srci[b])
            pltpu.sync_copy(dst_hbm.at[wid, pl.ds(g * G, G)], dsti[b])
            for j in range(G):
                pltpu.async_copy(y_hbm.at[srci[b].at[j]], rows[b].at[j],
                                 semg)
            _remap_dst(dsti[b], hoff)

        def drain_gather(b):
            for j in range(G):
                pltpu.make_async_copy(y_hbm.at[srci[b].at[j]],
                                      rows[b].at[j], semg).wait()

        def fire_scatter(b):
            for j in range(G):
                pltpu.async_copy(rows[b].at[j], acc.at[dsti[b].at[j]], sems,
                                 add=True)

        def drain_scatter(b):
            for j in range(G):
                pltpu.make_async_copy(rows[b].at[j], acc.at[dsti[b].at[j]],
                                      sems).wait()

        fire_gather(0, 0)

        def grp(k, carry):
            g = 2 * k
            fire_gather(g + 1, 1)
            drain_gather(0)
            fire_scatter(0)
            drain_scatter(0)
            fire_gather(g + 2, 0)
            drain_gather(1)
            fire_scatter(1)
            drain_scatter(1)
            return carry
        lax.fori_loop(0, (NG - 1) // 2, grp, None)
        drain_gather(0)
        fire_scatter(0)
        drain_scatter(0)
        plsc.subcore_barrier()
        _dump_acc(acc, out_hbm, c, s)
    return agg_kernel


_sc_aggregate_lo = _make_aggregate(0)
_sc_aggregate_hi = _make_aggregate(H)


def _stats_update(st_ref, u, first):
    part = jnp.concatenate(
        [jnp.sum(u, axis=0, keepdims=True),
         jnp.sum(u * u, axis=0, keepdims=True),
         jnp.zeros((6, EMB), jnp.float32)], axis=0)

    @pl.when(first)
    def _():
        st_ref[...] = part

    @pl.when(jnp.logical_not(first))
    def _():
        st_ref[...] = st_ref[...] + part


def _halves(h, lo0, lo1, hi0, hi1):
    return jnp.where(h == 0, lo0[0] + lo1[0], hi0[0] + hi1[0])


def _prep_body(dl0, dl1, dh0, dh1, nd, y0_ref, dd_ref):
    h = pl.program_id(0)
    cnt = _halves(h, dl0, dl1, dh0, dh1)[:, 0:1]
    deg = cnt + 1.0
    dinv = lax.rsqrt(deg)
    dgi = 1.0 / deg
    y0_ref[...] = nd[...] * dinv
    col = lax.broadcasted_iota(jnp.int32, (RB, EMB), 1)
    dd_ref[...] = jnp.where(col == 0, dinv, jnp.where(col == 1, dgi, 0.0))


_lo_spec = pl.BlockSpec(
    (1, RB, EMB), lambda h, i: (0, jnp.where(h == 0, i, HB - 1), 0))
_lo_spec1 = pl.BlockSpec(
    (1, RB, EMB), lambda h, i: (1, jnp.where(h == 0, i, HB - 1), 0))
_hi_spec = pl.BlockSpec(
    (1, RB, EMB), lambda h, i: (0, jnp.where(h == 0, 0, i), 0))
_hi_spec1 = pl.BlockSpec(
    (1, RB, EMB), lambda h, i: (1, jnp.where(h == 0, 0, i), 0))
_half_specs = [_lo_spec, _lo_spec1, _hi_spec, _hi_spec1]
_row2_spec = pl.BlockSpec((RB, EMB), lambda h, i: (h * HB + i, 0))
_st2_spec = pl.BlockSpec((8, EMB), lambda h, i: (0, 0))

_row_spec = pl.BlockSpec((RB, EMB), lambda i: (i, 0))
_st_spec = pl.BlockSpec((8, EMB), lambda i: (0, 0))


def _const_spec(shape, ndim=1):
    return pl.BlockSpec(shape, lambda *_: tuple(0 for _ in shape))


_tc_prep = pl.pallas_call(
    _prep_body,
    grid=(2, HB),
    in_specs=_half_specs + [_row2_spec],
    out_specs=[_row2_spec, _row2_spec],
    out_shape=[jax.ShapeDtypeStruct((N, EMB), jnp.float32),
               jax.ShapeDtypeStruct((N, EMB), jnp.float32)],
)


def _mix_body(sl0, sl1, sh0, sh1, x, dd, wg, bg, u_ref, st_ref):
    h = pl.program_id(0)
    i = pl.program_id(1)
    sv = _halves(h, sl0, sl1, sh0, sh1)
    ddv = dd[...]
    agg = sv * ddv[:, 0:1] + x[...] * ddv[:, 1:2]
    hh = jnp.maximum(
        jnp.dot(agg, wg[...], preferred_element_type=jnp.float32) + bg[...],
        0.0)
    u = hh + x[...]
    u_ref[...] = u
    _stats_update(st_ref, u, (h == 0) & (i == 0))


_tc_mix = pl.pallas_call(
    _mix_body,
    grid=(2, HB),
    in_specs=_half_specs + [_row2_spec, _row2_spec,
                            _const_spec((EMB, EMB)), _const_spec((1, EMB))],
    out_specs=[_row2_spec, _st2_spec],
    out_shape=[jax.ShapeDtypeStruct((N, EMB), jnp.float32),
               jax.ShapeDtypeStruct((8, EMB), jnp.float32)],
)


def _ff_body(u, p, w1, b1, w2, u2_ref, st_ref):
    pv = p[...]
    xp = u[...] * pv[0:1, :] + pv[1:2, :]
    h = jnp.maximum(
        jnp.dot(xp, w1[...], preferred_element_type=jnp.float32) + b1[...],
        0.0)
    u2 = jnp.dot(h, w2[...], preferred_element_type=jnp.float32) + xp
    u2_ref[...] = u2
    _stats_update(st_ref, u2, pl.program_id(0) == 0)


_tc_ff = pl.pallas_call(
    _ff_body,
    grid=(NB,),
    in_specs=[_row_spec, _const_spec((8, EMB)),
              _const_spec((EMB, MULT * EMB)), _const_spec((1, MULT * EMB)),
              _const_spec((MULT * EMB, EMB))],
    out_specs=[_row_spec, _st_spec],
    out_shape=[jax.ShapeDtypeStruct((N, EMB), jnp.float32),
               jax.ShapeDtypeStruct((8, EMB), jnp.float32)],
)


def _bn_body(u2, p, dd, x_ref, y_ref):
    pv = p[...]
    xv = u2[...] * pv[0:1, :] + pv[1:2, :]
    x_ref[...] = xv
    y_ref[...] = xv * dd[...][:, 0:1]


_tc_bn = pl.pallas_call(
    _bn_body,
    grid=(NB,),
    in_specs=[_row_spec, _const_spec((8, EMB)), _row_spec],
    out_specs=[_row_spec, _row_spec],
    out_shape=[jax.ShapeDtypeStruct((N, EMB), jnp.float32),
               jax.ShapeDtypeStruct((N, EMB), jnp.float32)],
)


def _cls_body(u2, p, cw, cb, o_ref):
    pv = p[...]
    xv = u2[...] * pv[0:1, :] + pv[1:2, :]
    o_ref[...] = (jnp.dot(xv, cw[...], preferred_element_type=jnp.float32)
                  + cb[...])


_tc_cls = pl.pallas_call(
    _cls_body,
    grid=(NB,),
    in_specs=[_row_spec, _const_spec((8, EMB)),
              _const_spec((EMB, NUMCLS)), _const_spec((1, NUMCLS))],
    out_specs=pl.BlockSpec((RB, NUMCLS), lambda i: (i, 0)),
    out_shape=jax.ShapeDtypeStruct((N, NUMCLS), jnp.float32),
)


def _bn_params(st, g, b):
    m = st[0] / N
    v = st[1] / N - m * m
    a = g / jnp.sqrt(v + 1e-5)
    cc = b - m * a
    return jnp.concatenate(
        [a[None], cc[None], jnp.zeros((6, EMB), jnp.float32)], axis=0)


def kernel(edge_index, nodes, Wg, bg, bn1_g, bn1_b, W1, b1, W2, bn2_g, bn2_b,
           cls_W, cls_b):
    src = edge_index[0]
    dst = edge_index[1]
    pad = jnp.arange(EPAD, dtype=jnp.int32)
    src_p = jnp.concatenate([src, pad % N]).reshape(NW, C, CH)
    dst_p = jnp.concatenate(
        [dst, jnp.full((EPAD,), 1 << 30, jnp.int32)]).reshape(NW, C, CH)

    deg_lo = _sc_degree_lo(dst_p)
    deg_hi = _sc_degree_hi(dst_p)
    y, dd = _tc_prep(deg_lo, deg_lo, deg_hi, deg_hi, nodes)

    x = nodes
    for i in range(DEPTH):
        s_lo = _sc_aggregate_lo(src_p, dst_p, y)
        s_hi = _sc_aggregate_hi(src_p, dst_p, y)
        u1, st1 = _tc_mix(s_lo, s_lo, s_hi, s_hi, x, dd, Wg[i], bg[i][None])
        p1 = _bn_params(st1, bn1_g[i], bn1_b[i])
        u2, st2 = _tc_ff(u1, p1, W1[i], b1[i][None], W2[i])
        p2 = _bn_params(st2, bn2_g[i], bn2_b[i])
        if i < DEPTH - 1:
            x, y = _tc_bn(u2, p2, dd)
        else:
            return _tc_cls(u2, p2, cls_W, cls_b[None])


# speedup vs baseline: 1.3937x; 1.3937x over previous
import functools

import jax
import jax.numpy as jnp
from jax import lax
from jax.experimental import pallas as pl
from jax.experimental.pallas import tpu as pltpu
from jax.experimental.pallas import tpu_sc as plsc

N = 100000
E = 3200000
EMB = 16
MULT = 4
DEPTH = 2
NUMCLS = 40

NC = 2
NS = 16
NW = NC * NS
CH = 128
G = 16
NG = 49
C = G * NG
EPW = C * CH
EPAD = NW * EPW - E

H = N // 2
AH = 50176
NDUM = AH - H
TRH = AH // NS
LTR = H - (NS - 1) * TRH
ZB = 392

HO = 51200
HPAD = HO - H
HPP = HO // 8
NPP = 2 * HPP
NREAL = H // 8
PB = 800
HB = HPP // PB
PB2 = 1600
NB = NPP // PB2
LN = 8 * EMB

_mesh = plsc.VectorSubcoreMesh(core_axis_name="c", subcore_axis_name="s")
_sc_params = pltpu.CompilerParams(use_tc_tiling_on_sc=False)


def _zero_acc(zbuf, acc, s):
    def zb(i, carry):
        zbuf[i] = jnp.zeros((EMB,), jnp.float32)
        return carry
    lax.fori_loop(0, ZB, zb, None)
    base = s * TRH
    for k in range(TRH // ZB):
        pltpu.sync_copy(zbuf, acc.at[pl.ds(base + k * ZB, ZB)])


def _remap_dst(dsti_b, hoff):
    for j in range(G):
        for k in range(CH // 16):
            sl = pl.ds(k * 16, 16)
            v = dsti_b[j, sl]
            off = v - hoff
            ok = (off >= 0) & (off < H)
            dum = (H + ((j * (CH // 16) + k) % (NDUM // 16)) * 16
                   + lax.iota(jnp.int32, 16))
            dsti_b[j, sl] = jnp.where(ok, off, dum)


def _dump_acc(acc, out_hbm, c, s, zbuf):
    base = s * TRH

    @pl.when(s == NS - 1)
    def _():
        pltpu.sync_copy(acc.at[pl.ds((NS - 1) * TRH, LTR)],
                        out_hbm.at[c, pl.ds((NS - 1) * TRH, LTR)])

    @pl.when(s != NS - 1)
    def _():
        pltpu.sync_copy(acc.at[pl.ds(base, TRH)],
                        out_hbm.at[c, pl.ds(base, TRH)])

    @pl.when(s == 0)
    def _():
        for k in range(HPAD // ZB):
            pltpu.sync_copy(zbuf, out_hbm.at[c, pl.ds(H + k * ZB, ZB)])
        rem = HPAD % ZB
        if rem:
            pltpu.sync_copy(
                zbuf.at[pl.ds(0, rem)],
                out_hbm.at[c, pl.ds(H + (HPAD // ZB) * ZB, rem)])


def _make_degree(hoff):
    @functools.partial(
        pl.kernel,
        out_type=jax.ShapeDtypeStruct((NC, HO, EMB), jnp.float32),
        mesh=_mesh,
        scratch_types=[
            pltpu.VMEM((G, CH), jnp.int32),
            pltpu.VMEM((G, CH), jnp.int32),
            pltpu.VMEM((CH, EMB), jnp.float32),
            pltpu.VMEM((ZB, EMB), jnp.float32),
            pltpu.VMEM_SHARED((AH, EMB), jnp.float32),
            pltpu.SemaphoreType.DMA,
        ],
        compiler_params=_sc_params,
    )
    def deg_kernel(dst_hbm, out_hbm, dsti0, dsti1, ones, zbuf, acc, sems):
        c = lax.axis_index("c")
        s = lax.axis_index("s")
        wid = c * NS + s
        dsti = [dsti0, dsti1]

        def ob(i, carry):
            ones[i] = jnp.ones((EMB,), jnp.float32)
            return carry
        lax.fori_loop(0, CH, ob, None)
        _zero_acc(zbuf, acc, s)
        plsc.subcore_barrier()

        def load(g, b):
            pltpu.sync_copy(dst_hbm.at[wid, pl.ds(g * G, G)], dsti[b])
            _remap_dst(dsti[b], hoff)

        def fire(b):
            for j in range(G):
                pltpu.async_copy(ones, acc.at[dsti[b].at[j]], sems, add=True)

        def drain(b):
            for j in range(G):
                pltpu.make_async_copy(ones, acc.at[dsti[b].at[j]],
                                      sems).wait()

        load(0, 0)

        def grp(k, carry):
            g = 2 * k
            fire(0)
            load(g + 1, 1)
            drain(0)
            fire(1)
            load(g + 2, 0)
            drain(1)
            return carry
        lax.fori_loop(0, (NG - 1) // 2, grp, None)
        fire(0)
        drain(0)
        plsc.subcore_barrier()
        _dump_acc(acc, out_hbm, c, s, zbuf)
    return deg_kernel


_sc_degree_lo = _make_degree(0)
_sc_degree_hi = _make_degree(H)


def _make_aggregate(hoff):
    @functools.partial(
        pl.kernel,
        out_type=jax.ShapeDtypeStruct((NC, HO, EMB), jnp.float32),
        mesh=_mesh,
        scratch_types=[
            pltpu.VMEM((G, CH), jnp.int32),
            pltpu.VMEM((G, CH), jnp.int32),
            pltpu.VMEM((G, CH), jnp.int32),
            pltpu.VMEM((G, CH), jnp.int32),
            pltpu.VMEM((G, CH, EMB), jnp.float32),
            pltpu.VMEM((G, CH, EMB), jnp.float32),
            pltpu.VMEM((ZB, EMB), jnp.float32),
            pltpu.VMEM_SHARED((AH, EMB), jnp.float32),
            pltpu.SemaphoreType.DMA,
            pltpu.SemaphoreType.DMA,
        ],
        compiler_params=_sc_params,
    )
    def agg_kernel(src_hbm, dst_hbm, y_hbm, out_hbm, srci0, srci1, dsti0,
                   dsti1, rows0, rows1, zbuf, acc, semg, sems):
        c = lax.axis_index("c")
        s = lax.axis_index("s")
        wid = c * NS + s
        srci = [srci0, srci1]
        dsti = [dsti0, dsti1]
        rows = [rows0, rows1]
        _zero_acc(zbuf, acc, s)
        plsc.subcore_barrier()

        def fire_gather(g, b):
            pltpu.sync_copy(src_hbm.at[wid, pl.ds(g * G, G)], srci[b])
            pltpu.sync_copy(dst_hbm.at[wid, pl.ds(g * G, G)], dsti[b])
            for k in range(CH // 16):
                sl = pl.ds(k * 16, 16)
                for j in range(G):
                    v = srci[b][j, sl]
                    srci[b][j, sl] = jnp.where(v >= H, v + HPAD, v)
            for j in range(G):
                pltpu.async_copy(y_hbm.at[srci[b].at[j]], rows[b].at[j],
                                 semg)
            _remap_dst(dsti[b], hoff)

        def drain_gather(b):
            for j in range(G):
                pltpu.make_async_copy(y_hbm.at[srci[b].at[j]],
                                      rows[b].at[j], semg).wait()

        def fire_scatter(b):
            for j in range(G):
                pltpu.async_copy(rows[b].at[j], acc.at[dsti[b].at[j]], sems,
                                 add=True)

        def drain_scatter(b):
            for j in range(G):
                pltpu.make_async_copy(rows[b].at[j], acc.at[dsti[b].at[j]],
                                      sems).wait()

        fire_gather(0, 0)

        def grp(k, carry):
            g = 2 * k
            fire_gather(g + 1, 1)
            drain_gather(0)
            fire_scatter(0)
            drain_scatter(0)
            fire_gather(g + 2, 0)
            drain_gather(1)
            fire_scatter(1)
            drain_scatter(1)
            return carry
        lax.fori_loop(0, (NG - 1) // 2, grp, None)
        drain_gather(0)
        fire_scatter(0)
        drain_scatter(0)
        plsc.subcore_barrier()
        _dump_acc(acc, out_hbm, c, s, zbuf)
    return agg_kernel


_sc_aggregate_lo = _make_aggregate(0)
_sc_aggregate_hi = _make_aggregate(H)


def _stats_update(st_ref, u, first):
    part = jnp.concatenate(
        [jnp.sum(u, axis=0, keepdims=True),
         jnp.sum(u * u, axis=0, keepdims=True),
         jnp.zeros((6, LN), jnp.float32)], axis=0)

    @pl.when(first)
    def _():
        st_ref[...] = part

    @pl.when(jnp.logical_not(first))
    def _():
        st_ref[...] = st_ref[...] + part


def _halves(h, lo0, lo1, hi0, hi1):
    return jnp.where(h == 0, lo0[0] + lo1[0], hi0[0] + hi1[0])


def _prep_body(dl0, dl1, dh0, dh1, nd, bc, y0_ref, di_ref, dg_ref):
    h = pl.program_id(0)
    cnt = jnp.dot(_halves(h, dl0, dl1, dh0, dh1), bc[...],
                  preferred_element_type=jnp.float32)
    deg = cnt + 1.0
    dinv = lax.rsqrt(deg)
    dgi = 1.0 / deg
    y0_ref[...] = nd[...] * dinv
    di_ref[...] = dinv
    dg_ref[...] = dgi


_lo_spec = pl.BlockSpec(
    (1, PB, LN), lambda h, i: (0, jnp.where(h == 0, i, HB - 1), 0))
_lo_spec1 = pl.BlockSpec(
    (1, PB, LN), lambda h, i: (1, jnp.where(h == 0, i, HB - 1), 0))
_hi_spec = pl.BlockSpec(
    (1, PB, LN), lambda h, i: (0, jnp.where(h == 0, 0, i), 0))
_hi_spec1 = pl.BlockSpec(
    (1, PB, LN), lambda h, i: (1, jnp.where(h == 0, 0, i), 0))
_half_specs = [_lo_spec, _lo_spec1, _hi_spec, _hi_spec1]
_row2_spec = pl.BlockSpec((PB, LN), lambda h, i: (h * HB + i, 0))
_st2_spec = pl.BlockSpec((8, LN), lambda h, i: (0, 0))

_row_spec = pl.BlockSpec((PB2, LN), lambda i: (i, 0))
_st_spec = pl.BlockSpec((8, LN), lambda i: (0, 0))


def _const_spec(shape):
    return pl.BlockSpec(shape, lambda *_: tuple(0 for _ in shape))


_tc_prep = pl.pallas_call(
    _prep_body,
    grid=(2, HB),
    in_specs=_half_specs + [_row2_spec, _const_spec((LN, LN))],
    out_specs=[_row2_spec, _row2_spec, _row2_spec],
    out_shape=[jax.ShapeDtypeStruct((NPP, LN), jnp.float32),
               jax.ShapeDtypeStruct((NPP, LN), jnp.float32),
               jax.ShapeDtypeStruct((NPP, LN), jnp.float32)],
)


def _mix_body(sl0, sl1, sh0, sh1, x, di, dg, wg, bg, u_ref, st_ref):
    h = pl.program_id(0)
    i = pl.program_id(1)
    sv = _halves(h, sl0, sl1, sh0, sh1)
    agg = sv * di[...] + x[...] * dg[...]
    hh = jnp.maximum(
        jnp.dot(agg, wg[...], preferred_element_type=jnp.float32) + bg[...],
        0.0)
    u = hh + x[...]
    u_ref[...] = u
    rmask = (lax.broadcasted_iota(jnp.int32, (PB, 1), 0) + i * PB) < NREAL
    _stats_update(st_ref, jnp.where(rmask, u, 0.0), (h == 0) & (i == 0))


_tc_mix = pl.pallas_call(
    _mix_body,
    grid=(2, HB),
    in_specs=_half_specs + [_row2_spec, _row2_spec, _row2_spec,
                            _const_spec((LN, LN)), _const_spec((1, LN))],
    out_specs=[_row2_spec, _st2_spec],
    out_shape=[jax.ShapeDtypeStruct((NPP, LN), jnp.float32),
               jax.ShapeDtypeStruct((8, LN), jnp.float32)],
)


def _ff_body(u, p, w1, b1, w2, u2_ref, st_ref):
    pv = p[...]
    xp = u[...] * pv[0:1, :] + pv[1:2, :]
    h = jnp.maximum(
        jnp.dot(xp, w1[...], preferred_element_type=jnp.float32) + b1[...],
        0.0)
    u2 = jnp.dot(h, w2[...], preferred_element_type=jnp.float32) + xp
    u2_ref[...] = u2
    g = lax.broadcasted_iota(jnp.int32, (PB2, 1), 0) + pl.program_id(0) * PB2
    rmask = lax.rem(g, HPP) < NREAL
    _stats_update(st_ref, jnp.where(rmask, u2, 0.0), pl.program_id(0) == 0)


_tc_ff = pl.pallas_call(
    _ff_body,
    grid=(NB,),
    in_specs=[_row_spec, _const_spec((8, LN)),
              _const_spec((LN, 8 * MULT * EMB)),
              _const_spec((1, 8 * MULT * EMB)),
              _const_spec((8 * MULT * EMB, LN))],
    out_specs=[_row_spec, _st_spec],
    out_shape=[jax.ShapeDtypeStruct((NPP, LN), jnp.float32),
               jax.ShapeDtypeStruct((8, LN), jnp.float32)],
)


def _bn_body(u2, p, di, x_ref, y_ref):
    pv = p[...]
    xv = u2[...] * pv[0:1, :] + pv[1:2, :]
    x_ref[...] = xv
    y_ref[...] = xv * di[...]


_tc_bn = pl.pallas_call(
    _bn_body,
    grid=(NB,),
    in_specs=[_row_spec, _const_spec((8, LN)), _row_spec],
    out_specs=[_row_spec, _row_spec],
    out_shape=[jax.ShapeDtypeStruct((NPP, LN), jnp.float32),
               jax.ShapeDtypeStruct((NPP, LN), jnp.float32)],
)


def _cls_body(u2, p, cw, cb, o_ref):
    pv = p[...]
    xv = u2[...] * pv[0:1, :] + pv[1:2, :]
    o_ref[...] = (jnp.dot(xv, cw[...], preferred_element_type=jnp.float32)
                  + cb[...])


_tc_cls = pl.pallas_call(
    _cls_body,
    grid=(NB,),
    in_specs=[_row_spec, _const_spec((8, LN)),
              _const_spec((LN, 8 * NUMCLS)), _const_spec((1, 8 * NUMCLS))],
    out_specs=pl.BlockSpec((PB2, 8 * NUMCLS), lambda i: (i, 0)),
    out_shape=jax.ShapeDtypeStruct((NPP, 8 * NUMCLS), jnp.float32),
)


def _bn_params(st, g, b):
    m = st[0].reshape(8, EMB).sum(0) / N
    sq = st[1].reshape(8, EMB).sum(0) / N
    v = sq - m * m
    a = g / jnp.sqrt(v + 1e-5)
    cc = b - m * a
    return jnp.concatenate(
        [jnp.tile(a, 8)[None], jnp.tile(cc, 8)[None],
         jnp.zeros((6, LN), jnp.float32)], axis=0)


import numpy as np

_BC = np.kron(np.eye(8, dtype=np.float32),
              np.outer((np.arange(EMB) == 0).astype(np.float32),
                       np.ones((EMB,), np.float32)))


def _kron8(w):
    return jnp.kron(jnp.asarray(np.eye(8, dtype=np.float32)), w)


def kernel(edge_index, nodes, Wg, bg, bn1_g, bn1_b, W1, b1, W2, bn2_g, bn2_b,
           cls_W, cls_b):
    src = edge_index[0]
    dst = edge_index[1]
    pad = jnp.arange(EPAD, dtype=jnp.int32)
    src_p = jnp.concatenate([src, pad % N]).reshape(NW, C, CH)
    dst_p = jnp.concatenate(
        [dst, jnp.full((EPAD,), 1 << 30, jnp.int32)]).reshape(NW, C, CH)

    deg_lo = _sc_degree_lo(dst_p).reshape(NC, HPP, LN)
    deg_hi = _sc_degree_hi(dst_p).reshape(NC, HPP, LN)
    zpad = jnp.zeros((HPP - NREAL, LN), jnp.float32)
    x = jnp.concatenate([nodes[:H].reshape(NREAL, LN), zpad,
                         nodes[H:].reshape(NREAL, LN), zpad])
    yp, di, dg = _tc_prep(deg_lo, deg_lo, deg_hi, deg_hi, x, _BC)

    for i in range(DEPTH):
        y = yp.reshape(2 * HO, EMB)
        s_lo = _sc_aggregate_lo(src_p, dst_p, y).reshape(NC, HPP, LN)
        s_hi = _sc_aggregate_hi(src_p, dst_p, y).reshape(NC, HPP, LN)
        u1, st1 = _tc_mix(s_lo, s_lo, s_hi, s_hi, x, di, dg,
                          _kron8(Wg[i]), jnp.tile(bg[i], 8)[None])
        p1 = _bn_params(st1, bn1_g[i], bn1_b[i])
        u2, st2 = _tc_ff(u1, p1, _kron8(W1[i]), jnp.tile(b1[i], 8)[None],
                         _kron8(W2[i]))
        p2 = _bn_params(st2, bn2_g[i], bn2_b[i])
        if i < DEPTH - 1:
            x, yp = _tc_bn(u2, p2, di)
        else:
            out = _tc_cls(u2, p2, _kron8(cls_W), jnp.tile(cls_b, 8)[None])
            out = jnp.concatenate([out[:NREAL], out[HPP:HPP + NREAL]])
            return out.reshape(N, NUMCLS)
